# baseline (speedup 1.0000x reference)
import functools

import jax
import jax.numpy as jnp
import numpy as np
from jax import lax
from jax.experimental import pallas as pl
from jax.experimental.pallas import tpu as pltpu, tpu_sc as plsc

_VOCAB = 100000
_D = 1024
_BLOCK = 2048
_BATCH = 4

_NC = 2
_NS = 16
_NW = _NC * _NS
_L = 16

_P_PER_W = _BLOCK // _NW
_C = 16
_NPC = _P_PER_W // _C


def _positional_encoding(length, d_model):
    pos = np.arange(length, dtype=np.float32)[:, np.newaxis]
    i = np.arange(d_model, dtype=np.float32)[np.newaxis, :]
    angle_rates = 1.0 / np.power(
        10000.0, (2.0 * np.floor(i / 2.0)) / np.float32(d_model))
    angle_rads = pos * angle_rates
    angle_rads[:, 0::2] = np.sin(angle_rads[:, 0::2])
    angle_rads[:, 1::2] = np.cos(angle_rads[:, 1::2])
    return angle_rads


_PE = jnp.asarray(_positional_encoding(_BLOCK, _D), dtype=jnp.float32)


def _body(x_hbm, pe_hbm, w_hbm, out_hbm,
          idx0, idx1, idx2, idx3, pe_v, r0, r1, r2, r3, sem):
    idx = (idx0, idx1, idx2, idx3)
    rows = (r0, r1, r2, r3)
    wid = lax.axis_index("s") * _NC + lax.axis_index("c")
    pos0 = wid * _P_PER_W

    @pl.loop(0, _NPC)
    def _chunk(pc):
        base = pos0 + pc * _C
        pltpu.sync_copy(pe_hbm.at[pl.ds(base, _C), :], pe_v)
        for b in range(_BATCH):
            pltpu.sync_copy(x_hbm.at[b, pl.ds(base, _C)], idx[b])
        descs = [pltpu.async_copy(w_hbm.at[idx[b]], rows[b], sem)
                 for b in range(_BATCH)]
        for d in descs:
            d.wait()

        @pl.loop(0, _C)
        def _row(rr):
            @pl.loop(0, _D // _L, unroll=8)
            def _col(cc):
                sl = pl.ds(cc * _L, _L)
                pe = pe_v[rr, sl]
                for b in range(_BATCH):
                    plsc.addupdate(rows[b].at[rr, sl], pe)

        for b in range(_BATCH):
            pltpu.sync_copy(rows[b], out_hbm.at[b, pl.ds(base, _C), :])


@jax.jit
def _run(x, pe, w):
    mesh = plsc.VectorSubcoreMesh(core_axis_name="c", subcore_axis_name="s")
    f = pl.kernel(
        _body,
        out_type=jax.ShapeDtypeStruct((_BATCH, _BLOCK, _D), jnp.float32),
        mesh=mesh,
        scratch_types=(
            [pltpu.VMEM((_C,), jnp.int32) for _ in range(_BATCH)]
            + [pltpu.VMEM((_C, _D), jnp.float32)]
            + [pltpu.VMEM((_C, _D), jnp.float32) for _ in range(_BATCH)]
            + [pltpu.SemaphoreType.DMA]
        ),
    )
    return f(x, pe, w)


def kernel(x, W):
    return _run(x.astype(jnp.int32), _PE, W)


# speedup vs baseline: 1.5169x; 1.5169x over previous
import jax
import jax.numpy as jnp
import numpy as np
from jax import lax
from jax.experimental import pallas as pl
from jax.experimental.pallas import tpu as pltpu, tpu_sc as plsc

_VOCAB = 100000
_D = 1024
_BLOCK = 2048
_BATCH = 4

_NC = 2
_NS = 16
_NW = _NC * _NS
_L = 16

_P_PER_W = _BLOCK // _NW
_C = 8
_NPC = _P_PER_W // _C
_R = _BATCH * _C


def _positional_encoding(length, d_model):
    pos = np.arange(length, dtype=np.float32)[:, np.newaxis]
    i = np.arange(d_model, dtype=np.float32)[np.newaxis, :]
    angle_rates = 1.0 / np.power(
        10000.0, (2.0 * np.floor(i / 2.0)) / np.float32(d_model))
    angle_rads = pos * angle_rates
    angle_rads[:, 0::2] = np.sin(angle_rads[:, 0::2])
    angle_rads[:, 1::2] = np.cos(angle_rads[:, 1::2])
    return angle_rads


_PE = jnp.asarray(_positional_encoding(_BLOCK, _D), dtype=jnp.float32)


def _body(x_hbm, pe_hbm, w_hbm, out_hbm,
          idx_v, pe0, pe1, rows0, rows1, gsem0, gsem1, wsem0, wsem1):
    pe_v = (pe0, pe1)
    rows = (rows0, rows1)
    gsem = (gsem0, gsem1)
    wsem = (wsem0, wsem1)
    wid = lax.axis_index("s") * _NC + lax.axis_index("c")
    pos0 = wid * _P_PER_W

    for b in range(_BATCH):
        pltpu.sync_copy(x_hbm.at[b, pl.ds(pos0, _P_PER_W)], idx_v.at[b])

    def issue_inputs(pc, slot):
        base = pos0 + pc * _C
        descs = [pltpu.async_copy(
            pe_hbm.at[pl.ds(base, _C), :], pe_v[slot], gsem[slot])]
        for b in range(_BATCH):
            descs.append(pltpu.async_copy(
                w_hbm.at[idx_v.at[b, pl.ds(pc * _C, _C)]],
                rows[slot].at[pl.ds(b * _C, _C), :], gsem[slot]))
        return descs

    def issue_writes(pc, slot):
        base = pos0 + pc * _C
        return [pltpu.async_copy(
            rows[slot].at[pl.ds(b * _C, _C), :],
            out_hbm.at[b, pl.ds(base, _C), :], wsem[slot])
            for b in range(_BATCH)]

    in_descs = {0: issue_inputs(0, 0)}
    out_descs = {}
    for pc in range(_NPC):
        slot = pc % 2
        nxt = (pc + 1) % 2
        for d in in_descs.pop(pc):
            d.wait()
        if pc + 1 < _NPC:
            if pc >= 1:
                for d in out_descs.pop(pc - 1):
                    d.wait()
            in_descs[pc + 1] = issue_inputs(pc + 1, nxt)

        @pl.loop(0, _C)
        def _row(rr):
            @pl.loop(0, _D // _L, unroll=8)
            def _col(cc):
                sl = pl.ds(cc * _L, _L)
                pe = pe_v[slot][rr, sl]
                for b in range(_BATCH):
                    plsc.addupdate(rows[slot].at[b * _C + rr, sl], pe)

        out_descs[pc] = issue_writes(pc, slot)

    for pc, descs in out_descs.items():
        for d in descs:
            d.wait()


@jax.jit
def _run(x, pe, w):
    mesh = plsc.VectorSubcoreMesh(core_axis_name="c", subcore_axis_name="s")
    f = pl.kernel(
        _body,
        out_type=jax.ShapeDtypeStruct((_BATCH, _BLOCK, _D), jnp.float32),
        mesh=mesh,
        scratch_types=(
            [pltpu.VMEM((_BATCH, _P_PER_W), jnp.int32)]
            + [pltpu.VMEM((_C, _D), jnp.float32) for _ in range(2)]
            + [pltpu.VMEM((_R, _D), jnp.float32) for _ in range(2)]
            + [pltpu.SemaphoreType.DMA for _ in range(4)]
        ),
    )
    return f(x, pe, w)


def kernel(x, W):
    return _run(x.astype(jnp.int32), _PE, W)
